# Initial kernel scaffold; baseline (speedup 1.0000x reference)
import jax
import jax.numpy as jnp
from jax.experimental import pallas as pl


def kernel(source_nodes, destination_nodes, edge_times, edge_idxs, edge_features, memory, last_update, time_w, time_b, W_ih, W_hh, b_ih, b_hh, W1, b1, W2, b2):
    raise NotImplementedError("write your pallas kernel here")


# speedup vs baseline: 3.2250x; 3.2250x over previous
import functools

import jax
import jax.numpy as jnp
from jax import lax
from jax.experimental import pallas as pl
from jax.experimental.pallas import tpu as pltpu

N_USERS = 10000
N_EDGES = 320000
D_EDGE = 16
MEM_DIM = 172
B = 20000
HID = 128
NUM_CLASSES = 2

NP = 10240
BP = 20480
MP = 176
ROW_BLK = 1024


def _dense_tc_kernel(mem_ref, mo_ref, ef_ref, dt_ref,
                     vr_m, vr_mo, vr_ef, vr_t,
                     vz_m, vz_mo, vz_ef, vz_t,
                     wn_m, wn_mo, wn_ef, wn_t,
                     whn, w1t, w2t,
                     br, bz, bn, bhn, b1, b2, tw, tb,
                     out_ref):
    def mm(a, b):
        return lax.dot_general(a, b, (((1,), (0,)), ((), ())),
                               preferred_element_type=jnp.float32)

    m = mem_ref[...]
    mo = mo_ref[...]
    ef = ef_ref[...]
    tenc = jnp.cos(dt_ref[...] * tw[...] + tb[...])

    ar = mm(m, vr_m[...]) + mm(mo, vr_mo[...]) + mm(ef, vr_ef[...]) \
        + mm(tenc, vr_t[...]) + br[...]
    az = mm(m, vz_m[...]) + mm(mo, vz_mo[...]) + mm(ef, vz_ef[...]) \
        + mm(tenc, vz_t[...]) + bz[...]
    i_n = mm(m, wn_m[...]) + mm(mo, wn_mo[...]) + mm(ef, wn_ef[...]) \
        + mm(tenc, wn_t[...]) + bn[...]
    h_n = mm(m, whn[...]) + bhn[...]

    r = jax.nn.sigmoid(ar)
    z = jax.nn.sigmoid(az)
    n = jnp.tanh(i_n + r * h_n)
    h_new = (1.0 - z) * n + z * m

    h = jnp.maximum(mm(h_new, w1t[...]) + b1[...], 0.0)
    out_ref[...] = mm(h, w2t[...]) + b2[...]


def _dense_stage(mem_p, mo, efg, dt, weights):
    grid = NP // ROW_BLK
    row_bs = lambda c: pl.BlockSpec((ROW_BLK, c), lambda i: (i, 0))
    const_bs = lambda shp: pl.BlockSpec(shp, lambda i: (0, 0))
    (vr_m, vr_mo, vr_ef, vr_t, vz_m, vz_mo, vz_ef, vz_t,
     wn_m, wn_mo, wn_ef, wn_t, whn, w1t, w2t,
     br, bz, bn, bhn, b1, b2, tw, tb) = weights
    in_specs = [row_bs(MP), row_bs(MP), row_bs(D_EDGE), row_bs(1)] + \
        [const_bs(w.shape) for w in weights]
    return pl.pallas_call(
        _dense_tc_kernel,
        grid=(grid,),
        in_specs=in_specs,
        out_specs=pl.BlockSpec((ROW_BLK, 16), lambda i: (i, 0)),
        out_shape=jax.ShapeDtypeStruct((NP, 16), jnp.float32),
    )(mem_p, mo, efg, dt, *weights)


def _prep_weights(W_ih, W_hh, b_ih, b_hh, W1, b1, W2, b2, time_w, time_b):
    D = MEM_DIM

    def padw(w):
        k, n = w.shape
        kp = -k % 16
        np_ = -n % 16
        return jnp.pad(w, ((0, kp), (0, np_)))

    def gate(w_rows):
        wm = w_rows[:, 0:D].T
        wmo = w_rows[:, D:2 * D].T
        wef = w_rows[:, 2 * D:2 * D + D_EDGE].T
        wt = w_rows[:, 2 * D + D_EDGE:].T
        return wm, wmo, wef, wt

    wr_m, wr_mo, wr_ef, wr_t = gate(W_ih[0:D])
    wz_m, wz_mo, wz_ef, wz_t = gate(W_ih[D:2 * D])
    wn_m, wn_mo, wn_ef, wn_t = gate(W_ih[2 * D:3 * D])
    whr = W_hh[0:D].T
    whz = W_hh[D:2 * D].T
    whn = W_hh[2 * D:3 * D].T

    vr_m = padw(wr_m + whr)
    vz_m = padw(wz_m + whz)
    row = lambda v, n: jnp.pad(v, (0, -v.shape[0] % n)).reshape(1, -1)
    weights = (
        vr_m, padw(wr_mo), padw(wr_ef), padw(wr_t),
        vz_m, padw(wz_mo), padw(wz_ef), padw(wz_t),
        padw(wn_m), padw(wn_mo), padw(wn_ef), padw(wn_t),
        padw(whn), padw(W1.T), padw(W2.T),
        row(b_ih[0:D] + b_hh[0:D], 16),
        row(b_ih[D:2 * D] + b_hh[D:2 * D], 16),
        row(b_ih[2 * D:3 * D], 16),
        row(b_hh[2 * D:3 * D], 16),
        row(b1, 16), row(b2, 16),
        row(time_w, 16), row(time_b, 16),
    )
    return weights


def kernel(source_nodes, destination_nodes, edge_times, edge_idxs,
           edge_features, memory, last_update, time_w, time_b,
           W_ih, W_hh, b_ih, b_hh, W1, b1, W2, b2):
    src = source_nodes.astype(jnp.int32)
    dst = destination_nodes.astype(jnp.int32)
    eidx = edge_idxs.astype(jnp.int32)

    pad_node = NP - 1
    src_p = jnp.pad(src, (0, BP - B), constant_values=pad_node)
    dst_p = jnp.pad(dst, (0, BP - B), constant_values=pad_node)
    et_p = jnp.pad(edge_times, (0, BP - B))
    ei_p = jnp.pad(eidx, (0, BP - B))
    mem_p = jnp.pad(memory, ((0, NP - N_USERS), (0, MP - MEM_DIM)))
    lu_p = jnp.pad(last_update, (0, NP - N_USERS))

    all_nodes = jnp.concatenate([src_p, dst_p])
    order = jnp.arange(2 * BP, dtype=jnp.int32)
    pos = jnp.full((NP,), -1, jnp.int32).at[all_nodes].max(order)

    p0 = jnp.maximum(pos, 0)
    side = p0 >= BP
    e = p0 - jnp.where(side, BP, 0)
    other = jnp.where(side, src_p[e], dst_p[e])
    eidx_g = ei_p[e]
    dt = et_p[e] - lu_p

    mo = mem_p[other]
    efg = edge_features[eidx_g]

    weights = _prep_weights(W_ih, W_hh, b_ih, b_hh, W1, b1, W2, b2,
                            time_w, time_b)
    logits_node = _dense_stage(mem_p, mo, efg, dt[:, None], weights)

    logits = logits_node[src]

    return logits[:, :NUM_CLASSES]
